# baseline (speedup 1.0000x reference)
import functools

import jax
import jax.numpy as jnp
from jax import lax
from jax.experimental import pallas as pl
from jax.experimental.pallas import tpu as pltpu
from jax.experimental.pallas import tpu_sc as plsc

N_NODES = 10000
N_EDGES = 320000
D = 128

NC = 2
NS = 16
NW = NC * NS

B_EDGE = 128
CHUNKS = 80
E_PAD = NW * CHUNKS * B_EDGE
W_IDX = 16
NWIN = CHUNKS // W_IDX

NPAD = 10240
ROWS_PER_TILE = NPAD // NS
ZR = 128


def _zero_fill(buf, rows, cols):
    def body(r, carry):
        for c in range(cols // 16):
            buf[r, pl.ds(c * 16, 16)] = jnp.zeros((16,), jnp.float32)
        return carry
    lax.fori_loop(0, rows, body, 0, unroll=False)


@functools.cache
def _sc_mesh():
    return plsc.VectorSubcoreMesh(
        core_axis_name="c", subcore_axis_name="s", num_cores=NC, num_subcores=NS)


@functools.cache
def _sc_degree_kernel():
    return pl.kernel(
        _sc_degree,
        out_type=jax.ShapeDtypeStruct((NC, NPAD), jnp.float32),
        mesh=_sc_mesh(),
        scratch_types=[
            pltpu.VMEM((CHUNKS, B_EDGE), jnp.int32),
            pltpu.VMEM((ZR,), jnp.float32),
            pltpu.VMEM((ZR,), jnp.float32),
            pltpu.VMEM_SHARED((NPAD,), jnp.float32),
            pltpu.SemaphoreType.DMA,
        ],
    )


_DEG_Q = 8


def _sc_degree(dst_hbm, out_hbm, dst_v, ones_v, zeros_v, acc, sem):
    cid = lax.axis_index("c")
    sid = lax.axis_index("s")
    wid = cid * NS + sid
    fetch = pltpu.async_copy(dst_hbm.at[wid], dst_v, sem)
    for c in range(ZR // 16):
        ones_v[pl.ds(c * 16, 16)] = jnp.ones((16,), jnp.float32)
        zeros_v[pl.ds(c * 16, 16)] = jnp.zeros((16,), jnp.float32)
    for k in range(ROWS_PER_TILE // ZR):
        pltpu.sync_copy(zeros_v, acc.at[pl.ds(sid * ROWS_PER_TILE + k * ZR, ZR)])
    fetch.wait()
    plsc.subcore_barrier()

    def body(g, carry):
        for k in range(_DEG_Q):
            pltpu.async_copy(ones_v.at[pl.ds(0, B_EDGE)],
                             acc.at[dst_v.at[g * _DEG_Q + k]], sem, add=True)
        for k in range(_DEG_Q):
            pltpu.make_async_copy(ones_v.at[pl.ds(0, B_EDGE)],
                                  acc.at[dst_v.at[g * _DEG_Q + k]], sem).wait()
        return carry

    lax.fori_loop(0, CHUNKS // _DEG_Q, body, 0, unroll=False)
    plsc.subcore_barrier()
    pltpu.sync_copy(acc.at[pl.ds(sid * ROWS_PER_TILE, ROWS_PER_TILE)],
                    out_hbm.at[cid, pl.ds(sid * ROWS_PER_TILE, ROWS_PER_TILE)])


@functools.cache
def _sc_aggregate_kernel():
    return pl.kernel(
        _sc_aggregate,
        out_type=jax.ShapeDtypeStruct((NC, NPAD, D), jnp.float32),
        mesh=_sc_mesh(),
        scratch_types=[
            pltpu.VMEM((CHUNKS, B_EDGE), jnp.int32),
            pltpu.VMEM((W_IDX, B_EDGE), jnp.int32),
            pltpu.VMEM((W_IDX, B_EDGE), jnp.int32),
            pltpu.VMEM((B_EDGE, D), jnp.float32),
            pltpu.VMEM((B_EDGE, D), jnp.float32),
            pltpu.VMEM_SHARED((NPAD, D), jnp.float32),
            pltpu.SemaphoreType.DMA,
            pltpu.SemaphoreType.DMA,
            pltpu.SemaphoreType.DMA,
        ],
    )


def _sc_aggregate(hs_hbm, src_hbm, dst_hbm, out_hbm,
                  src_v, dstw0, dstw1, rows0, rows1, acc, sem0, sem1, semw):
    cid = lax.axis_index("c")
    sid = lax.axis_index("s")
    wid = cid * NS + sid
    src_fetch = pltpu.async_copy(src_hbm.at[wid], src_v, sem0)
    dst_fetch = pltpu.async_copy(dst_hbm.at[wid, pl.ds(0, W_IDX)], dstw0, semw)
    _zero_fill(rows0, B_EDGE, D)
    base = sid * ROWS_PER_TILE
    for k in range(ROWS_PER_TILE // B_EDGE):
        pltpu.sync_copy(rows0, acc.at[pl.ds(base + k * B_EDGE, B_EDGE)])
    src_fetch.wait()
    dst_fetch.wait()
    plsc.subcore_barrier()

    pltpu.async_copy(hs_hbm.at[src_v.at[0]], rows0, sem0)
    dst_bufs = (dstw0, dstw1)
    for w in range(NWIN):
        cur = dst_bufs[w % 2]
        nxt = dst_bufs[(w + 1) % 2]
        if w + 1 < NWIN:
            fetch = pltpu.async_copy(
                dst_hbm.at[wid, pl.ds((w + 1) * W_IDX, W_IDX)], nxt, semw)

        def body(i, carry, cur=cur, wbase=w * W_IDX):
            jj = wbase + 2 * i
            pltpu.async_copy(hs_hbm.at[src_v.at[jj + 1]], rows1, sem1)
            pltpu.make_async_copy(hs_hbm.at[src_v.at[jj]], rows0, sem0).wait()
            pltpu.sync_copy(rows0, acc.at[cur.at[2 * i]], add=True)

            @pl.when(jj + 2 < CHUNKS)
            def _():
                pltpu.async_copy(hs_hbm.at[src_v.at[jj + 2]], rows0, sem0)

            pltpu.make_async_copy(hs_hbm.at[src_v.at[jj + 1]], rows1, sem1).wait()
            pltpu.sync_copy(rows1, acc.at[cur.at[2 * i + 1]], add=True)
            return carry

        lax.fori_loop(0, W_IDX // 2, body, 0, unroll=False)
        if w + 1 < NWIN:
            fetch.wait()

    plsc.subcore_barrier()
    pltpu.sync_copy(acc.at[pl.ds(sid * ROWS_PER_TILE, ROWS_PER_TILE)],
                    out_hbm.at[cid, pl.ds(sid * ROWS_PER_TILE, ROWS_PER_TILE)])


BLK = 2000


def _tc_prep_body(p0_ref, p1_ref, x_ref, dinv_ref, xs_ref):
    deg = p0_ref[0] + p1_ref[0] + 1.0
    dinv = lax.rsqrt(deg)
    dinv_ref[...] = dinv
    xs_ref[...] = dinv * x_ref[...]


def _tc_layer1_body(t0_ref, t1_ref, xs_ref, dinv_ref, w_ref, b_ref, hs_ref):
    dinv = dinv_ref[...]
    agg = dinv * (t0_ref[0] + t1_ref[0] + xs_ref[...])
    pre = jnp.dot(agg, w_ref[...], preferred_element_type=jnp.float32) + b_ref[...]
    h = jnp.where(pre > 0, pre, jnp.exp(pre) - 1.0)
    hs_ref[...] = dinv * h


def _tc_final_body(t0_ref, t1_ref, hs_ref, dinv_ref, wmu_ref, bmu_ref,
                   wlv_ref, blv_ref, eps_ref, z_ref, mu_ref, lv_ref):
    agg = dinv_ref[...] * (t0_ref[0] + t1_ref[0] + hs_ref[...])
    mu = jnp.dot(agg, wmu_ref[...], preferred_element_type=jnp.float32) + bmu_ref[...]
    lv = jnp.dot(agg, wlv_ref[...], preferred_element_type=jnp.float32) + blv_ref[...]
    mu_ref[...] = mu
    lv_ref[...] = lv
    z_ref[...] = mu + eps_ref[...] * jnp.exp(0.5 * lv)


def _row_spec(cols):
    return pl.BlockSpec((BLK, cols), lambda i: (i, 0))


def _part_spec(core, cols):
    return pl.BlockSpec((1, BLK, cols), lambda i, c=core: (c, i, 0))


def _const_spec(shape):
    return pl.BlockSpec(shape, lambda i: (0, 0))


def kernel(x, edge_index, W1, b1, W_mu, b_mu, W_lv, b_lv):
    per_tile = N_EDGES // NW
    pad = CHUNKS * B_EDGE - per_tile
    pad_src = jnp.broadcast_to(jnp.arange(pad, dtype=jnp.int32), (NW, pad))
    pad_dst = jnp.broadcast_to(
        N_NODES + jnp.arange(pad, dtype=jnp.int32), (NW, pad))
    src_row = lax.optimization_barrier(edge_index)[0].astype(jnp.int32)
    src = jnp.concatenate(
        [src_row.reshape(NW, per_tile), pad_src], axis=1
    ).reshape(NW, CHUNKS, B_EDGE)
    dst = jnp.concatenate(
        [edge_index[1].astype(jnp.int32).reshape(NW, per_tile), pad_dst], axis=1
    ).reshape(NW, CHUNKS, B_EDGE)

    grid = N_NODES // BLK

    deg_parts = _sc_degree_kernel()(dst).reshape(NC, NPAD, 1)

    dinv, xs = pl.pallas_call(
        _tc_prep_body,
        grid=(grid,),
        in_specs=[_part_spec(0, 1), _part_spec(1, 1), _row_spec(D)],
        out_specs=[_row_spec(1), _row_spec(D)],
        out_shape=[
            jax.ShapeDtypeStruct((N_NODES, 1), jnp.float32),
            jax.ShapeDtypeStruct((N_NODES, D), jnp.float32),
        ],
    )(deg_parts, deg_parts, x)

    t_parts = _sc_aggregate_kernel()(xs, src, dst)

    hs = pl.pallas_call(
        _tc_layer1_body,
        grid=(grid,),
        in_specs=[_part_spec(0, D), _part_spec(1, D), _row_spec(D), _row_spec(1),
                  _const_spec((D, D)), _const_spec((1, D))],
        out_specs=_row_spec(D),
        out_shape=jax.ShapeDtypeStruct((N_NODES, D), jnp.float32),
    )(t_parts, t_parts, xs, dinv, W1, b1.reshape(1, D))

    t2_parts = _sc_aggregate_kernel()(hs, src, dst)

    with jax.ensure_compile_time_eval():
        eps = jax.random.normal(jax.random.key(1234), (N_NODES, D), jnp.float32)
    z, mu, logvar = pl.pallas_call(
        _tc_final_body,
        grid=(grid,),
        in_specs=[_part_spec(0, D), _part_spec(1, D), _row_spec(D), _row_spec(1),
                  _const_spec((D, D)), _const_spec((1, D)),
                  _const_spec((D, D)), _const_spec((1, D)), _row_spec(D)],
        out_specs=[_row_spec(D), _row_spec(D), _row_spec(D)],
        out_shape=[
            jax.ShapeDtypeStruct((N_NODES, D), jnp.float32),
            jax.ShapeDtypeStruct((N_NODES, D), jnp.float32),
            jax.ShapeDtypeStruct((N_NODES, D), jnp.float32),
        ],
    )(t2_parts, t2_parts, hs, dinv, W_mu, b_mu.reshape(1, D),
      W_lv, b_lv.reshape(1, D), eps)

    return (z, mu, logvar)


# speedup vs baseline: 1.0315x; 1.0315x over previous
import functools

import jax
import jax.numpy as jnp
from jax import lax
from jax.experimental import pallas as pl
from jax.experimental.pallas import tpu as pltpu
from jax.experimental.pallas import tpu_sc as plsc

N_NODES = 10000
N_EDGES = 320000
D = 128

NC = 2
NS = 16
NW = NC * NS

B_EDGE = 128
CHUNKS = 80
E_PAD = NW * CHUNKS * B_EDGE
W_IDX = 16
NWIN = CHUNKS // W_IDX

NPAD = 10240
ROWS_PER_TILE = NPAD // NS
ZR = 128


def _zero_fill(buf, rows, cols):
    def body(r, carry):
        for c in range(cols // 16):
            buf[r, pl.ds(c * 16, 16)] = jnp.zeros((16,), jnp.float32)
        return carry
    lax.fori_loop(0, rows, body, 0, unroll=False)


@functools.cache
def _sc_mesh():
    return plsc.VectorSubcoreMesh(
        core_axis_name="c", subcore_axis_name="s", num_cores=NC, num_subcores=NS)


@functools.cache
def _sc_degree_kernel():
    return pl.kernel(
        _sc_degree,
        out_type=jax.ShapeDtypeStruct((NC, NPAD), jnp.float32),
        mesh=_sc_mesh(),
        scratch_types=[
            pltpu.VMEM((CHUNKS, B_EDGE), jnp.int32),
            pltpu.VMEM((ZR,), jnp.float32),
            pltpu.VMEM((ZR,), jnp.float32),
            pltpu.VMEM_SHARED((NPAD,), jnp.float32),
            pltpu.SemaphoreType.DMA,
        ],
    )


_DEG_Q = 8


def _sc_degree(dst_hbm, out_hbm, dst_v, ones_v, zeros_v, acc, sem):
    cid = lax.axis_index("c")
    sid = lax.axis_index("s")
    wid = cid * NS + sid
    fetch = pltpu.async_copy(dst_hbm.at[wid], dst_v, sem)
    for c in range(ZR // 16):
        ones_v[pl.ds(c * 16, 16)] = jnp.ones((16,), jnp.float32)
        zeros_v[pl.ds(c * 16, 16)] = jnp.zeros((16,), jnp.float32)
    for k in range(ROWS_PER_TILE // ZR):
        pltpu.sync_copy(zeros_v, acc.at[pl.ds(sid * ROWS_PER_TILE + k * ZR, ZR)])
    fetch.wait()
    plsc.subcore_barrier()

    def body(g, carry):
        for k in range(_DEG_Q):
            pltpu.async_copy(ones_v.at[pl.ds(0, B_EDGE)],
                             acc.at[dst_v.at[g * _DEG_Q + k]], sem, add=True)
        for k in range(_DEG_Q):
            pltpu.make_async_copy(ones_v.at[pl.ds(0, B_EDGE)],
                                  acc.at[dst_v.at[g * _DEG_Q + k]], sem).wait()
        return carry

    lax.fori_loop(0, CHUNKS // _DEG_Q, body, 0, unroll=False)
    plsc.subcore_barrier()
    pltpu.sync_copy(acc.at[pl.ds(sid * ROWS_PER_TILE, ROWS_PER_TILE)],
                    out_hbm.at[cid, pl.ds(sid * ROWS_PER_TILE, ROWS_PER_TILE)])


@functools.cache
def _sc_aggregate_kernel():
    return pl.kernel(
        _sc_aggregate,
        out_type=jax.ShapeDtypeStruct((NC, NPAD, D), jnp.float32),
        mesh=_sc_mesh(),
        scratch_types=[
            pltpu.VMEM((CHUNKS, B_EDGE), jnp.int32),
            pltpu.VMEM((W_IDX, B_EDGE), jnp.int32),
            pltpu.VMEM((W_IDX, B_EDGE), jnp.int32),
            pltpu.VMEM((B_EDGE, D), jnp.float32),
            pltpu.VMEM((B_EDGE, D), jnp.float32),
            pltpu.VMEM_SHARED((NPAD, D), jnp.float32),
            pltpu.SemaphoreType.DMA,
            pltpu.SemaphoreType.DMA,
            pltpu.SemaphoreType.DMA,
        ],
    )


def _sc_aggregate(hs_hbm, src_hbm, dst_hbm, out_hbm,
                  src_v, dstw0, dstw1, rows0, rows1, acc, sem0, sem1, semw):
    cid = lax.axis_index("c")
    sid = lax.axis_index("s")
    wid = cid * NS + sid
    src_fetch = pltpu.async_copy(src_hbm.at[wid], src_v, sem0)
    dst_fetch = pltpu.async_copy(dst_hbm.at[wid, pl.ds(0, W_IDX)], dstw0, semw)
    _zero_fill(rows0, B_EDGE, D)
    base = sid * ROWS_PER_TILE
    for k in range(ROWS_PER_TILE // B_EDGE):
        pltpu.sync_copy(rows0, acc.at[pl.ds(base + k * B_EDGE, B_EDGE)])
    src_fetch.wait()
    dst_fetch.wait()
    plsc.subcore_barrier()

    pltpu.async_copy(hs_hbm.at[src_v.at[0]], rows0, sem0)
    dst_bufs = (dstw0, dstw1)
    for w in range(NWIN):
        cur = dst_bufs[w % 2]
        nxt = dst_bufs[(w + 1) % 2]
        if w + 1 < NWIN:
            fetch = pltpu.async_copy(
                dst_hbm.at[wid, pl.ds((w + 1) * W_IDX, W_IDX)], nxt, semw)

        def body(i, carry, cur=cur, wbase=w * W_IDX):
            jj = wbase + 2 * i
            pltpu.async_copy(hs_hbm.at[src_v.at[jj + 1]], rows1, sem1)
            pltpu.make_async_copy(hs_hbm.at[src_v.at[jj]], rows0, sem0).wait()
            pltpu.sync_copy(rows0, acc.at[cur.at[2 * i]], add=True)

            @pl.when(jj + 2 < CHUNKS)
            def _():
                pltpu.async_copy(hs_hbm.at[src_v.at[jj + 2]], rows0, sem0)

            pltpu.make_async_copy(hs_hbm.at[src_v.at[jj + 1]], rows1, sem1).wait()
            pltpu.sync_copy(rows1, acc.at[cur.at[2 * i + 1]], add=True)
            return carry

        lax.fori_loop(0, W_IDX // 2, body, 0, unroll=False)
        if w + 1 < NWIN:
            fetch.wait()

    plsc.subcore_barrier()
    pltpu.sync_copy(acc.at[pl.ds(sid * ROWS_PER_TILE, ROWS_PER_TILE)],
                    out_hbm.at[cid, pl.ds(sid * ROWS_PER_TILE, ROWS_PER_TILE)])


BLK = 2000


def _tc_prep_body(p0_ref, p1_ref, x_ref, dinv_ref, xs_ref):
    deg = p0_ref[0] + p1_ref[0] + 1.0
    dinv = lax.rsqrt(deg)
    dinv_ref[...] = dinv
    xs_ref[...] = dinv * x_ref[...]


def _tc_layer1_body(t0_ref, t1_ref, xs_ref, dinv_ref, w_ref, b_ref, hs_ref):
    dinv = dinv_ref[...]
    agg = dinv * (t0_ref[0] + t1_ref[0] + xs_ref[...])
    pre = jnp.dot(agg, w_ref[...], preferred_element_type=jnp.float32) + b_ref[...]
    h = jnp.where(pre > 0, pre, jnp.exp(pre) - 1.0)
    hs_ref[...] = dinv * h


def _tc_final_body(t0_ref, t1_ref, hs_ref, dinv_ref, wmu_ref, bmu_ref,
                   wlv_ref, blv_ref, eps_ref, z_ref, mu_ref, lv_ref):
    agg = dinv_ref[...] * (t0_ref[0] + t1_ref[0] + hs_ref[...])
    mu = jnp.dot(agg, wmu_ref[...], preferred_element_type=jnp.float32) + bmu_ref[...]
    lv = jnp.dot(agg, wlv_ref[...], preferred_element_type=jnp.float32) + blv_ref[...]
    mu_ref[...] = mu
    lv_ref[...] = lv
    z_ref[...] = mu + eps_ref[...] * jnp.exp(0.5 * lv)


def _row_spec(cols):
    return pl.BlockSpec((BLK, cols), lambda i: (i, 0))


def _part_spec(core, cols):
    return pl.BlockSpec((1, BLK, cols), lambda i, c=core: (c, i, 0))


def _const_spec(shape):
    return pl.BlockSpec(shape, lambda i: (0, 0))


def kernel(x, edge_index, W1, b1, W_mu, b_mu, W_lv, b_lv):
    per_tile = N_EDGES // NW
    pad = CHUNKS * B_EDGE - per_tile
    pad_src = jnp.broadcast_to(jnp.arange(pad, dtype=jnp.int32), (NW, pad))
    pad_dst = jnp.broadcast_to(
        N_NODES + jnp.arange(pad, dtype=jnp.int32), (NW, pad))
    src_row = lax.optimization_barrier(edge_index[0].astype(jnp.int32))
    src = jnp.concatenate(
        [src_row.reshape(NW, per_tile), pad_src], axis=1
    ).reshape(NW, CHUNKS, B_EDGE)
    dst = jnp.concatenate(
        [edge_index[1].astype(jnp.int32).reshape(NW, per_tile), pad_dst], axis=1
    ).reshape(NW, CHUNKS, B_EDGE)

    grid = N_NODES // BLK

    deg_parts = _sc_degree_kernel()(dst).reshape(NC, NPAD, 1)

    dinv, xs = pl.pallas_call(
        _tc_prep_body,
        grid=(grid,),
        in_specs=[_part_spec(0, 1), _part_spec(1, 1), _row_spec(D)],
        out_specs=[_row_spec(1), _row_spec(D)],
        out_shape=[
            jax.ShapeDtypeStruct((N_NODES, 1), jnp.float32),
            jax.ShapeDtypeStruct((N_NODES, D), jnp.float32),
        ],
    )(deg_parts, deg_parts, x)

    t_parts = _sc_aggregate_kernel()(xs, src, dst)

    hs = pl.pallas_call(
        _tc_layer1_body,
        grid=(grid,),
        in_specs=[_part_spec(0, D), _part_spec(1, D), _row_spec(D), _row_spec(1),
                  _const_spec((D, D)), _const_spec((1, D))],
        out_specs=_row_spec(D),
        out_shape=jax.ShapeDtypeStruct((N_NODES, D), jnp.float32),
    )(t_parts, t_parts, xs, dinv, W1, b1.reshape(1, D))

    t2_parts = _sc_aggregate_kernel()(hs, src, dst)

    with jax.ensure_compile_time_eval():
        eps = jax.random.normal(jax.random.key(1234), (N_NODES, D), jnp.float32)
    z, mu, logvar = pl.pallas_call(
        _tc_final_body,
        grid=(grid,),
        in_specs=[_part_spec(0, D), _part_spec(1, D), _row_spec(D), _row_spec(1),
                  _const_spec((D, D)), _const_spec((1, D)),
                  _const_spec((D, D)), _const_spec((1, D)), _row_spec(D)],
        out_specs=[_row_spec(D), _row_spec(D), _row_spec(D)],
        out_shape=[
            jax.ShapeDtypeStruct((N_NODES, D), jnp.float32),
            jax.ShapeDtypeStruct((N_NODES, D), jnp.float32),
            jax.ShapeDtypeStruct((N_NODES, D), jnp.float32),
        ],
    )(t2_parts, t2_parts, hs, dinv, W_mu, b_mu.reshape(1, D),
      W_lv, b_lv.reshape(1, D), eps)

    return (z, mu, logvar)
